# baseline (speedup 1.0000x reference)
import functools

import jax
import jax.numpy as jnp
from jax import lax
from jax.experimental import pallas as pl
from jax.experimental.pallas import tpu as pltpu
from jax.experimental.pallas import tpu_sc as plsc

N, E, D = 10000, 320000, 128
NC, NS = 2, 16
NW = NC * NS
EPW = E // NW
C = 128
NCHT = E // C
NST = -(-NCHT // NW)
NPAD = 10240
RPS = NPAD // NS
ZR = 80


def _sc_segment_sum(edge_index, x):
  mesh = plsc.VectorSubcoreMesh(core_axis_name="c", subcore_axis_name="s")

  @functools.partial(
      pl.kernel,
      out_type=jax.ShapeDtypeStruct((NC, NPAD, D), jnp.float32),
      mesh=mesh,
      scratch_types=[
          pltpu.VMEM((2, C), jnp.int32),
          pltpu.VMEM((2, C), jnp.int32),
          pltpu.VMEM((2, C), jnp.int32),
          pltpu.VMEM((2, C), jnp.int32),
          pltpu.VMEM((C, D), jnp.float32),
          pltpu.VMEM((C, D), jnp.float32),
          pltpu.VMEM((ZR, D), jnp.float32),
          pltpu.VMEM_SHARED((NPAD, D), jnp.float32),
          [pltpu.SemaphoreType.DMA] * 4,
          pltpu.SemaphoreType.DMA,
          pltpu.SemaphoreType.DMA,
      ],
  )
  def kern(edges_hbm, x_hbm, out_hbm, idx0, idx1, idx2, idx3,
           rows_a, rows_b, zbuf, acc, isems, gsem_a, gsem_b):
    cid = lax.axis_index("c")
    sid = lax.axis_index("s")
    wid = sid * NC + cid
    idxs = [idx0, idx1, idx2, idx3]

    def fetch_idx(t, slot):
      k = wid + NW * t
      pltpu.async_copy(
          edges_hbm.at[pl.ds(0, 2), pl.ds(k * C, C)],
          idxs[slot], isems[slot])

    def wait_idx(slot):
      pltpu.make_async_copy(edges_hbm.at[pl.ds(0, 2), pl.ds(0, C)],
                            idxs[slot], isems[slot]).wait()

    def gather(slot, buf, sem):
      pltpu.async_copy(x_hbm.at[idxs[slot].at[0]], buf, sem)

    def wait_rows(buf, sem):
      pltpu.make_async_copy(x_hbm.at[idx0.at[0]], buf, sem).wait()

    def scatter(slot, buf):
      pltpu.sync_copy(buf, acc.at[idxs[slot].at[1]], add=True)

    for t in range(4):
      fetch_idx(t, t)

    z16 = jnp.zeros((16,), jnp.float32)

    def zrow(i, _):
      for k in range(D // 16):
        zbuf[i, pl.ds(k * 16, 16)] = z16
      return 0

    lax.fori_loop(0, ZR, zrow, 0)
    wait_idx(0)
    gather(0, rows_a, gsem_a)
    wait_idx(1)
    gather(1, rows_b, gsem_b)
    for k in range(RPS // ZR):
      pltpu.sync_copy(zbuf, acc.at[pl.ds(sid * RPS + k * ZR, ZR)])
    plsc.subcore_barrier()

    def valid(t):
      return wid + NW * t < NCHT

    def quad(q, _):
      t0 = 4 * q
      stages = [(0, rows_a, gsem_a), (1, rows_b, gsem_b),
                (2, rows_a, gsem_a), (3, rows_b, gsem_b)]
      for j, (slot, rbuf, gsem) in enumerate(stages):
        t = t0 + j

        @pl.when(valid(t))
        def _():
          wait_rows(rbuf, gsem)
          scatter(slot, rbuf)

        @pl.when(valid(t + 4))
        def _():
          fetch_idx(t + 4, slot)

        @pl.when(valid(t + 2))
        def _():
          nslot = (slot + 2) % 4
          wait_idx(nslot)
          gather(nslot, rbuf, gsem)

      return 0

    lax.fori_loop(0, -(-NST // 4), quad, 0)
    plsc.subcore_barrier()

    pltpu.sync_copy(acc.at[pl.ds(sid * RPS, RPS)],
                    out_hbm.at[cid, pl.ds(sid * RPS, RPS)])

  return kern(edge_index, x)


def _tc_mlp(x, partial, W1, b1, W2, b2, eps):
  BR = 2000

  def kern(x_ref, p0_ref, p1_ref, w1_ref, b1_ref, w2_ref, b2_ref, eps_ref,
           out_ref):
    scale = 1.0 + eps_ref[0, 0]
    acc = scale * x_ref[...] + p0_ref[0] + p1_ref[0]
    h = jnp.dot(acc, w1_ref[...], preferred_element_type=jnp.float32)
    h = jnp.maximum(h + b1_ref[...], 0.0)
    o = jnp.dot(h, w2_ref[...], preferred_element_type=jnp.float32)
    out_ref[...] = o + b2_ref[...]

  row_spec = pl.BlockSpec((BR, D), lambda i: (i, 0))
  p0_spec = pl.BlockSpec((1, BR, D), lambda i: (0, i, 0))
  p1_spec = pl.BlockSpec((1, BR, D), lambda i: (1, i, 0))
  full = pl.BlockSpec((D, D), lambda i: (0, 0))
  vec = pl.BlockSpec((1, D), lambda i: (0, 0))
  return pl.pallas_call(
      kern,
      grid=(N // BR,),
      in_specs=[row_spec, p0_spec, p1_spec, full, vec, full, vec,
                pl.BlockSpec(memory_space=pltpu.SMEM)],
      out_specs=row_spec,
      out_shape=jax.ShapeDtypeStruct((N, D), jnp.float32),
  )(x, partial, partial, W1, b1, W2, b2, eps)


@jax.jit
def kernel(x, edge_index, W1, b1, W2, b2, eps):
  partial = _sc_segment_sum(edge_index, x)
  return _tc_mlp(x, partial, W1, b1.reshape(1, D), W2, b2.reshape(1, D),
                 eps)


# speedup vs baseline: 1.0046x; 1.0046x over previous
import functools

import jax
import jax.numpy as jnp
from jax import lax
from jax.experimental import pallas as pl
from jax.experimental.pallas import tpu as pltpu
from jax.experimental.pallas import tpu_sc as plsc

N, E, D = 10000, 320000, 128
NC, NS = 2, 16
NW = NC * NS
EPW = E // NW
C = 128
NCHT = E // C
NST = -(-NCHT // NW)
NPAD = 10240
RPS = NPAD // NS
ZR = 80


def _sc_segment_sum(edge_index, x):
  mesh = plsc.VectorSubcoreMesh(core_axis_name="c", subcore_axis_name="s")

  @functools.partial(
      pl.kernel,
      out_type=jax.ShapeDtypeStruct((NC, NPAD, D), jnp.float32),
      mesh=mesh,
      compiler_params=pltpu.CompilerParams(use_tc_tiling_on_sc=True),
      scratch_types=[
          pltpu.VMEM((2, C), jnp.int32),
          pltpu.VMEM((2, C), jnp.int32),
          pltpu.VMEM((2, C), jnp.int32),
          pltpu.VMEM((2, C), jnp.int32),
          pltpu.VMEM((C, D), jnp.float32),
          pltpu.VMEM((C, D), jnp.float32),
          pltpu.VMEM((ZR, D), jnp.float32),
          pltpu.VMEM_SHARED((NPAD, D), jnp.float32),
          [pltpu.SemaphoreType.DMA] * 4,
          pltpu.SemaphoreType.DMA,
          pltpu.SemaphoreType.DMA,
      ],
  )
  def kern(edges_hbm, x_hbm, out_hbm, idx0, idx1, idx2, idx3,
           rows_a, rows_b, zbuf, acc, isems, gsem_a, gsem_b):
    cid = lax.axis_index("c")
    sid = lax.axis_index("s")
    wid = sid * NC + cid
    idxs = [idx0, idx1, idx2, idx3]

    def fetch_idx(t, slot):
      k = wid + NW * t
      pltpu.async_copy(
          edges_hbm.at[pl.ds(0, 2), pl.ds(k * C, C)],
          idxs[slot], isems[slot])

    def wait_idx(slot):
      pltpu.make_async_copy(edges_hbm.at[pl.ds(0, 2), pl.ds(0, C)],
                            idxs[slot], isems[slot]).wait()

    def gather(slot, buf, sem):
      pltpu.async_copy(x_hbm.at[idxs[slot].at[0]], buf, sem)

    def wait_rows(buf, sem):
      pltpu.make_async_copy(x_hbm.at[idx0.at[0]], buf, sem).wait()

    def scatter(slot, buf):
      pltpu.sync_copy(buf, acc.at[idxs[slot].at[1]], add=True)

    for t in range(4):
      fetch_idx(t, t)

    z16 = jnp.zeros((16,), jnp.float32)

    def zrow(i, _):
      for k in range(D // 16):
        zbuf[i, pl.ds(k * 16, 16)] = z16
      return 0

    lax.fori_loop(0, ZR, zrow, 0)
    wait_idx(0)
    gather(0, rows_a, gsem_a)
    wait_idx(1)
    gather(1, rows_b, gsem_b)
    for k in range(RPS // ZR):
      pltpu.sync_copy(zbuf, acc.at[pl.ds(sid * RPS + k * ZR, ZR)])
    plsc.subcore_barrier()

    def valid(t):
      return wid + NW * t < NCHT

    def quad(q, _):
      t0 = 4 * q
      stages = [(0, rows_a, gsem_a), (1, rows_b, gsem_b),
                (2, rows_a, gsem_a), (3, rows_b, gsem_b)]
      for j, (slot, rbuf, gsem) in enumerate(stages):
        t = t0 + j

        @pl.when(valid(t))
        def _():
          wait_rows(rbuf, gsem)
          scatter(slot, rbuf)

        @pl.when(valid(t + 4))
        def _():
          fetch_idx(t + 4, slot)

        @pl.when(valid(t + 2))
        def _():
          nslot = (slot + 2) % 4
          wait_idx(nslot)
          gather(nslot, rbuf, gsem)

      return 0

    lax.fori_loop(0, -(-NST // 4), quad, 0)
    plsc.subcore_barrier()

    pltpu.sync_copy(acc.at[pl.ds(sid * RPS, RPS)],
                    out_hbm.at[cid, pl.ds(sid * RPS, RPS)])

  return kern(edge_index, x)


def _tc_mlp(x, partial, W1, b1, W2, b2, eps):
  BR = 2000

  def kern(x_ref, p0_ref, p1_ref, w1_ref, b1_ref, w2_ref, b2_ref, eps_ref,
           out_ref):
    scale = 1.0 + eps_ref[0, 0]
    acc = scale * x_ref[...] + p0_ref[0] + p1_ref[0]
    h = jnp.dot(acc, w1_ref[...], preferred_element_type=jnp.float32)
    h = jnp.maximum(h + b1_ref[...], 0.0)
    o = jnp.dot(h, w2_ref[...], preferred_element_type=jnp.float32)
    out_ref[...] = o + b2_ref[...]

  row_spec = pl.BlockSpec((BR, D), lambda i: (i, 0))
  p0_spec = pl.BlockSpec((1, BR, D), lambda i: (0, i, 0))
  p1_spec = pl.BlockSpec((1, BR, D), lambda i: (1, i, 0))
  full = pl.BlockSpec((D, D), lambda i: (0, 0))
  vec = pl.BlockSpec((1, D), lambda i: (0, 0))
  return pl.pallas_call(
      kern,
      grid=(N // BR,),
      in_specs=[row_spec, p0_spec, p1_spec, full, vec, full, vec,
                pl.BlockSpec(memory_space=pltpu.SMEM)],
      out_specs=row_spec,
      out_shape=jax.ShapeDtypeStruct((N, D), jnp.float32),
  )(x, partial, partial, W1, b1, W2, b2, eps)


@jax.jit
def kernel(x, edge_index, W1, b1, W2, b2, eps):
  partial = _sc_segment_sum(edge_index, x)
  return _tc_mlp(x, partial, W1, b1.reshape(1, D), W2, b2.reshape(1, D),
                 eps)


# speedup vs baseline: 1.0086x; 1.0040x over previous
import functools

import jax
import jax.numpy as jnp
from jax import lax
from jax.experimental import pallas as pl
from jax.experimental.pallas import tpu as pltpu
from jax.experimental.pallas import tpu_sc as plsc

N, E, D = 10000, 320000, 128
NC, NS = 2, 16
NW = NC * NS
EPW = E // NW
C = 128
NCHT = E // C
NST = -(-NCHT // NW)
NPAD = 10240
RPS = NPAD // NS
ZR = 80


def _sc_segment_sum(edge_index, x):
  mesh = plsc.VectorSubcoreMesh(core_axis_name="c", subcore_axis_name="s")

  @functools.partial(
      pl.kernel,
      out_type=jax.ShapeDtypeStruct((NC, NPAD, D), jnp.float32),
      mesh=mesh,
      compiler_params=pltpu.CompilerParams(use_tc_tiling_on_sc=True),
      scratch_types=[
          pltpu.VMEM((2, C), jnp.int32),
          pltpu.VMEM((2, C), jnp.int32),
          pltpu.VMEM((2, C), jnp.int32),
          pltpu.VMEM((2, C), jnp.int32),
          pltpu.VMEM((C, D), jnp.float32),
          pltpu.VMEM((C, D), jnp.float32),
          pltpu.VMEM((ZR, D), jnp.float32),
          pltpu.VMEM_SHARED((NPAD, D), jnp.float32),
          [pltpu.SemaphoreType.DMA] * 4,
          pltpu.SemaphoreType.DMA,
          pltpu.SemaphoreType.DMA,
      ],
  )
  def kern(edges_hbm, x_hbm, out_hbm, idx0, idx1, idx2, idx3,
           rows_a, rows_b, zbuf, acc, isems, gsem_a, gsem_b):
    cid = lax.axis_index("c")
    sid = lax.axis_index("s")
    wid = sid * NC + cid
    idxs = [idx0, idx1, idx2, idx3]

    def fetch_idx(t, slot):
      k = wid + NW * t
      pltpu.async_copy(
          edges_hbm.at[pl.ds(0, 2), pl.ds(k * C, C)],
          idxs[slot], isems[slot])

    def wait_idx(slot):
      pltpu.make_async_copy(edges_hbm.at[pl.ds(0, 2), pl.ds(0, C)],
                            idxs[slot], isems[slot]).wait()

    def gather(slot, buf, sem):
      pltpu.async_copy(x_hbm.at[idxs[slot].at[0]], buf, sem)

    def wait_rows(buf, sem):
      pltpu.make_async_copy(x_hbm.at[idx0.at[0]], buf, sem).wait()

    def scatter(slot, buf):
      pltpu.sync_copy(buf, acc.at[idxs[slot].at[1]], add=True)

    for t in range(4):
      fetch_idx(t, t)

    z16 = jnp.zeros((16,), jnp.float32)

    def zrow(i, _):
      for k in range(D // 16):
        zbuf[i, pl.ds(k * 16, 16)] = z16
      return 0

    lax.fori_loop(0, ZR, zrow, 0)
    wait_idx(0)
    gather(0, rows_a, gsem_a)
    wait_idx(1)
    gather(1, rows_b, gsem_b)
    for k in range(RPS // ZR):
      pltpu.sync_copy(zbuf, acc.at[pl.ds(sid * RPS + k * ZR, ZR)])
    plsc.subcore_barrier()

    def valid(t):
      return wid + NW * t < NCHT

    def quad(q, _):
      t0 = 4 * q
      stages = [(0, rows_a, gsem_a), (1, rows_b, gsem_b),
                (2, rows_a, gsem_a), (3, rows_b, gsem_b)]
      for j, (slot, rbuf, gsem) in enumerate(stages):
        t = t0 + j

        @pl.when(valid(t))
        def _():
          wait_rows(rbuf, gsem)
          scatter(slot, rbuf)

        @pl.when(valid(t + 4))
        def _():
          fetch_idx(t + 4, slot)

        @pl.when(valid(t + 2))
        def _():
          nslot = (slot + 2) % 4
          wait_idx(nslot)
          gather(nslot, rbuf, gsem)

      return 0

    lax.fori_loop(0, -(-NST // 4), quad, 0)
    plsc.subcore_barrier()

    pltpu.sync_copy(acc.at[pl.ds(sid * RPS, RPS)],
                    out_hbm.at[cid, pl.ds(sid * RPS, RPS)])

  return kern(edge_index, x)


def _tc_mlp(x, partial, W1, b1, W2, b2, eps):
  BR = 5000

  def kern(x_ref, p0_ref, p1_ref, w1_ref, b1_ref, w2_ref, b2_ref, eps_ref,
           out_ref):
    scale = 1.0 + eps_ref[0, 0]
    acc = scale * x_ref[...] + p0_ref[0] + p1_ref[0]
    h = jnp.dot(acc, w1_ref[...], preferred_element_type=jnp.float32)
    h = jnp.maximum(h + b1_ref[...], 0.0)
    o = jnp.dot(h, w2_ref[...], preferred_element_type=jnp.float32)
    out_ref[...] = o + b2_ref[...]

  row_spec = pl.BlockSpec((BR, D), lambda i: (i, 0))
  p0_spec = pl.BlockSpec((1, BR, D), lambda i: (0, i, 0))
  p1_spec = pl.BlockSpec((1, BR, D), lambda i: (1, i, 0))
  full = pl.BlockSpec((D, D), lambda i: (0, 0))
  vec = pl.BlockSpec((1, D), lambda i: (0, 0))
  return pl.pallas_call(
      kern,
      grid=(N // BR,),
      in_specs=[row_spec, p0_spec, p1_spec, full, vec, full, vec,
                pl.BlockSpec(memory_space=pltpu.SMEM)],
      out_specs=row_spec,
      out_shape=jax.ShapeDtypeStruct((N, D), jnp.float32),
  )(x, partial, partial, W1, b1, W2, b2, eps)


@jax.jit
def kernel(x, edge_index, W1, b1, W2, b2, eps):
  partial = _sc_segment_sum(edge_index, x)
  return _tc_mlp(x, partial, W1, b1.reshape(1, D), W2, b2.reshape(1, D),
                 eps)
